# baseline (speedup 1.0000x reference)
import jax
import jax.numpy as jnp
from jax.experimental import pallas as pl
from jax.experimental.pallas import tpu as pltpu

_B, _C, _S, _OUT = 64, 3, 448, 224


def _racnn_body(locs_ref, img_ref, out_ref):
    b = pl.program_id(0)
    fS = jnp.float32(_S)
    tx = locs_ref[b, 0]
    ty = locs_ref[b, 1]
    tl = locs_ref[b, 2]
    tl = jnp.clip(tl, fS / 3.0, fS * 2.0 / 3.0)
    tx = jnp.clip(tx, tl, fS - tl)
    ty = jnp.clip(ty, tl, fS - tl)
    w_off = jnp.maximum(jnp.floor(tx - tl), 0.0)
    w_end = jnp.where(tx + tl < fS, jnp.floor(tx + tl), fS)
    h_off = jnp.maximum(jnp.floor(ty - tl), 0.0)
    h_end = jnp.where(ty + tl < fS, jnp.floor(ty + tl), fS)

    def weights(off, end, shape, out_axis):
        r = jax.lax.broadcasted_iota(jnp.int32, shape, out_axis).astype(jnp.float32)
        s = jax.lax.broadcasted_iota(jnp.int32, shape, 1 - out_axis).astype(jnp.float32)
        L = end - off
        src = off + r * (L - 1.0) / (_OUT - 1.0)
        i0 = jnp.clip(jnp.floor(src), 0.0, fS - 1.0)
        i1 = jnp.minimum(i0 + 1.0, fS - 1.0)
        fr = src - i0
        w = jnp.where(s == i0, 1.0 - fr, 0.0) + jnp.where(s == i1, fr, 0.0)
        mask = jax.nn.sigmoid(10.0 * (s - off)) - jax.nn.sigmoid(10.0 * (s - end))
        return w * mask

    a_w = weights(w_off, w_end, (_OUT, _S), 0)
    b_w = weights(h_off, h_end, (_S, _OUT), 1)

    img = img_ref[0].astype(jnp.bfloat16)
    y = jnp.dot(img, b_w.astype(jnp.bfloat16),
                preferred_element_type=jnp.float32).astype(jnp.bfloat16)
    a_wb = a_w.astype(jnp.bfloat16)
    for c in range(_C):
        out_ref[0, c] = jnp.dot(a_wb, y[c * _S:(c + 1) * _S, :],
                                preferred_element_type=jnp.float32)


def kernel(images, locs):
    imgs2 = images.reshape(_B, _C * _S, _S)
    return pl.pallas_call(
        _racnn_body,
        grid=(_B,),
        in_specs=[
            pl.BlockSpec(memory_space=pltpu.SMEM),
            pl.BlockSpec((1, _C * _S, _S), lambda b: (b, 0, 0)),
        ],
        out_specs=pl.BlockSpec((1, _C, _OUT, _OUT), lambda b: (b, 0, 0, 0)),
        out_shape=jax.ShapeDtypeStruct((_B, _C, _OUT, _OUT), jnp.float32),
        compiler_params=pltpu.CompilerParams(
            dimension_semantics=("arbitrary",),
        ),
    )(locs, imgs2)


# speedup vs baseline: 1.0118x; 1.0118x over previous
import jax
import jax.numpy as jnp
from jax.experimental import pallas as pl
from jax.experimental.pallas import tpu as pltpu

_B, _C, _S, _OUT = 64, 3, 448, 224


def _racnn_body(locs_ref, img_ref, out_ref):
    b = pl.program_id(0)
    fS = jnp.float32(_S)
    tx = locs_ref[b, 0]
    ty = locs_ref[b, 1]
    tl = locs_ref[b, 2]
    tl = jnp.clip(tl, fS / 3.0, fS * 2.0 / 3.0)
    tx = jnp.clip(tx, tl, fS - tl)
    ty = jnp.clip(ty, tl, fS - tl)
    w_off = jnp.maximum(jnp.floor(tx - tl), 0.0)
    w_end = jnp.where(tx + tl < fS, jnp.floor(tx + tl), fS)
    h_off = jnp.maximum(jnp.floor(ty - tl), 0.0)
    h_end = jnp.where(ty + tl < fS, jnp.floor(ty + tl), fS)

    si = jax.lax.broadcasted_iota(jnp.int32, (_S, _OUT), 0)

    def weights(off, end):
        r = jax.lax.broadcasted_iota(jnp.int32, (1, _OUT), 1).astype(jnp.float32)
        L = end - off
        src = off + r * (L - 1.0) / (_OUT - 1.0)
        i0f = jnp.clip(jnp.floor(src), 0.0, fS - 1.0)
        i1f = jnp.minimum(i0f + 1.0, fS - 1.0)
        fr = src - i0f
        sig = jax.nn.sigmoid
        m0 = sig(10.0 * (i0f - off)) - sig(10.0 * (i0f - end))
        m1 = sig(10.0 * (i1f - off)) - sig(10.0 * (i1f - end))
        w0 = (1.0 - fr) * m0
        w1 = fr * m1
        i0 = i0f.astype(jnp.int32)
        i1 = i1f.astype(jnp.int32)
        w = jnp.where(si == i0, w0, 0.0) + jnp.where(si == i1, w1, 0.0)
        return w.astype(jnp.bfloat16)

    wx = weights(w_off, w_end)
    wy = weights(h_off, h_end)

    img = img_ref[0].astype(jnp.bfloat16)
    y = jnp.dot(img, wy, preferred_element_type=jnp.float32).astype(jnp.bfloat16)
    for c in range(_C):
        out_ref[0, c] = jax.lax.dot_general(
            wx, y[c * _S:(c + 1) * _S, :],
            ((( 0,), (0,)), ((), ())),
            preferred_element_type=jnp.float32)


def kernel(images, locs):
    imgs2 = images.reshape(_B, _C * _S, _S)
    return pl.pallas_call(
        _racnn_body,
        grid=(_B,),
        in_specs=[
            pl.BlockSpec(memory_space=pltpu.SMEM),
            pl.BlockSpec((1, _C * _S, _S), lambda b: (b, 0, 0)),
        ],
        out_specs=pl.BlockSpec((1, _C, _OUT, _OUT), lambda b: (b, 0, 0, 0)),
        out_shape=jax.ShapeDtypeStruct((_B, _C, _OUT, _OUT), jnp.float32),
        compiler_params=pltpu.CompilerParams(
            dimension_semantics=("arbitrary",),
        ),
    )(locs, imgs2)


# speedup vs baseline: 1.2554x; 1.2407x over previous
import jax
import jax.numpy as jnp
from jax.experimental import pallas as pl
from jax.experimental.pallas import tpu as pltpu

_B, _C, _S, _OUT = 64, 3, 448, 224


_BB = 2


def _racnn_one(locs_ref, img_ref, out_ref, b, k):
    fS = jnp.float32(_S)
    tx = locs_ref[b, 0]
    ty = locs_ref[b, 1]
    tl = locs_ref[b, 2]
    tl = jnp.clip(tl, fS / 3.0, fS * 2.0 / 3.0)
    tx = jnp.clip(tx, tl, fS - tl)
    ty = jnp.clip(ty, tl, fS - tl)
    w_off = jnp.maximum(jnp.floor(tx - tl), 0.0)
    w_end = jnp.where(tx + tl < fS, jnp.floor(tx + tl), fS)
    h_off = jnp.maximum(jnp.floor(ty - tl), 0.0)
    h_end = jnp.where(ty + tl < fS, jnp.floor(ty + tl), fS)

    si = jax.lax.broadcasted_iota(jnp.int32, (_S, _OUT), 0)

    def weights(off, end):
        r = jax.lax.broadcasted_iota(jnp.int32, (1, _OUT), 1).astype(jnp.float32)
        L = end - off
        src = off + r * (L - 1.0) / (_OUT - 1.0)
        i0f = jnp.clip(jnp.floor(src), 0.0, fS - 1.0)
        i1f = jnp.minimum(i0f + 1.0, fS - 1.0)
        fr = src - i0f
        sig = jax.nn.sigmoid
        m0 = sig(10.0 * (i0f - off)) - sig(10.0 * (i0f - end))
        m1 = sig(10.0 * (i1f - off)) - sig(10.0 * (i1f - end))
        w0 = (1.0 - fr) * m0
        w1 = fr * m1
        i0 = i0f.astype(jnp.int32)
        i1 = i1f.astype(jnp.int32)
        w = jnp.where(si == i0, w0, 0.0) + jnp.where(si == i1, w1, 0.0)
        return w.astype(jnp.bfloat16)

    wx = weights(w_off, w_end)
    wy = weights(h_off, h_end)

    img = img_ref[k].astype(jnp.bfloat16)
    y = jnp.dot(img, wy, preferred_element_type=jnp.float32).astype(jnp.bfloat16)
    for c in range(_C):
        out_ref[k, c] = jax.lax.dot_general(
            wx, y[c * _S:(c + 1) * _S, :],
            ((( 0,), (0,)), ((), ())),
            preferred_element_type=jnp.float32)


def _racnn_body(locs_ref, img_ref, out_ref):
    g = pl.program_id(0)
    for k in range(_BB):
        _racnn_one(locs_ref, img_ref, out_ref, g * _BB + k, k)


def kernel(images, locs):
    imgs2 = images.reshape(_B, _C * _S, _S)
    return pl.pallas_call(
        _racnn_body,
        grid=(_B // _BB,),
        in_specs=[
            pl.BlockSpec(memory_space=pltpu.SMEM),
            pl.BlockSpec((_BB, _C * _S, _S), lambda b: (b, 0, 0)),
        ],
        out_specs=pl.BlockSpec((_BB, _C, _OUT, _OUT), lambda b: (b, 0, 0, 0)),
        out_shape=jax.ShapeDtypeStruct((_B, _C, _OUT, _OUT), jnp.float32),
        compiler_params=pltpu.CompilerParams(
            dimension_semantics=("arbitrary",),
        ),
    )(locs, imgs2)


# speedup vs baseline: 1.4060x; 1.1199x over previous
import jax
import jax.numpy as jnp
from jax.experimental import pallas as pl
from jax.experimental.pallas import tpu as pltpu

_B, _C, _S, _OUT = 64, 3, 448, 224


_BB = 4


def _racnn_one(locs_ref, img_ref, out_ref, b, k):
    fS = jnp.float32(_S)
    tx = locs_ref[b, 0]
    ty = locs_ref[b, 1]
    tl = locs_ref[b, 2]
    tl = jnp.clip(tl, fS / 3.0, fS * 2.0 / 3.0)
    tx = jnp.clip(tx, tl, fS - tl)
    ty = jnp.clip(ty, tl, fS - tl)
    w_off = jnp.maximum(jnp.floor(tx - tl), 0.0)
    w_end = jnp.where(tx + tl < fS, jnp.floor(tx + tl), fS)
    h_off = jnp.maximum(jnp.floor(ty - tl), 0.0)
    h_end = jnp.where(ty + tl < fS, jnp.floor(ty + tl), fS)

    si = jax.lax.broadcasted_iota(jnp.int32, (_S, _OUT), 0)

    def weights(off, end):
        r = jax.lax.broadcasted_iota(jnp.int32, (1, _OUT), 1).astype(jnp.float32)
        L = end - off
        src = off + r * (L - 1.0) / (_OUT - 1.0)
        i0f = jnp.clip(jnp.floor(src), 0.0, fS - 1.0)
        i1f = jnp.minimum(i0f + 1.0, fS - 1.0)
        fr = src - i0f
        sig = jax.nn.sigmoid
        m0 = sig(10.0 * (i0f - off)) - sig(10.0 * (i0f - end))
        m1 = sig(10.0 * (i1f - off)) - sig(10.0 * (i1f - end))
        w0 = (1.0 - fr) * m0
        w1 = fr * m1
        i0 = i0f.astype(jnp.int32)
        i1 = i1f.astype(jnp.int32)
        w = jnp.where(si == i0, w0, 0.0) + jnp.where(si == i1, w1, 0.0)
        return w.astype(jnp.bfloat16)

    wx = weights(w_off, w_end)
    wy = weights(h_off, h_end)

    img = img_ref[k].astype(jnp.bfloat16)
    y = jnp.dot(img, wy, preferred_element_type=jnp.float32).astype(jnp.bfloat16)
    for c in range(_C):
        out_ref[k, c] = jax.lax.dot_general(
            wx, y[c * _S:(c + 1) * _S, :],
            ((( 0,), (0,)), ((), ())),
            preferred_element_type=jnp.float32)


def _racnn_body(locs_ref, img_ref, out_ref):
    g = pl.program_id(0)
    for k in range(_BB):
        _racnn_one(locs_ref, img_ref, out_ref, g * _BB + k, k)


def kernel(images, locs):
    imgs2 = images.reshape(_B, _C * _S, _S)
    return pl.pallas_call(
        _racnn_body,
        grid=(_B // _BB,),
        in_specs=[
            pl.BlockSpec(memory_space=pltpu.SMEM),
            pl.BlockSpec((_BB, _C * _S, _S), lambda b: (b, 0, 0)),
        ],
        out_specs=pl.BlockSpec((_BB, _C, _OUT, _OUT), lambda b: (b, 0, 0, 0)),
        out_shape=jax.ShapeDtypeStruct((_B, _C, _OUT, _OUT), jnp.float32),
        compiler_params=pltpu.CompilerParams(
            dimension_semantics=("arbitrary",),
        ),
    )(locs, imgs2)


# speedup vs baseline: 1.4301x; 1.0171x over previous
import jax
import jax.numpy as jnp
from jax.experimental import pallas as pl
from jax.experimental.pallas import tpu as pltpu

_B, _C, _S, _OUT = 64, 3, 448, 224


_BB = 8


def _racnn_one(locs_ref, img_ref, out_ref, b, k):
    fS = jnp.float32(_S)
    tx = locs_ref[b, 0]
    ty = locs_ref[b, 1]
    tl = locs_ref[b, 2]
    tl = jnp.clip(tl, fS / 3.0, fS * 2.0 / 3.0)
    tx = jnp.clip(tx, tl, fS - tl)
    ty = jnp.clip(ty, tl, fS - tl)
    w_off = jnp.maximum(jnp.floor(tx - tl), 0.0)
    w_end = jnp.where(tx + tl < fS, jnp.floor(tx + tl), fS)
    h_off = jnp.maximum(jnp.floor(ty - tl), 0.0)
    h_end = jnp.where(ty + tl < fS, jnp.floor(ty + tl), fS)

    si = jax.lax.broadcasted_iota(jnp.int32, (_S, _OUT), 0)

    def weights(off, end):
        r = jax.lax.broadcasted_iota(jnp.int32, (1, _OUT), 1).astype(jnp.float32)
        L = end - off
        src = off + r * (L - 1.0) / (_OUT - 1.0)
        i0f = jnp.clip(jnp.floor(src), 0.0, fS - 1.0)
        i1f = jnp.minimum(i0f + 1.0, fS - 1.0)
        fr = src - i0f
        sig = jax.nn.sigmoid
        m0 = sig(10.0 * (i0f - off)) - sig(10.0 * (i0f - end))
        m1 = sig(10.0 * (i1f - off)) - sig(10.0 * (i1f - end))
        w0 = (1.0 - fr) * m0
        w1 = fr * m1
        i0 = i0f.astype(jnp.int32)
        i1 = i1f.astype(jnp.int32)
        w = jnp.where(si == i0, w0, 0.0) + jnp.where(si == i1, w1, 0.0)
        return w.astype(jnp.bfloat16)

    wx = weights(w_off, w_end)
    wy = weights(h_off, h_end)

    img = img_ref[k].astype(jnp.bfloat16)
    y = jnp.dot(img, wy, preferred_element_type=jnp.float32).astype(jnp.bfloat16)
    for c in range(_C):
        out_ref[k, c] = jax.lax.dot_general(
            wx, y[c * _S:(c + 1) * _S, :],
            ((( 0,), (0,)), ((), ())),
            preferred_element_type=jnp.float32)


def _racnn_body(locs_ref, img_ref, out_ref):
    g = pl.program_id(0)
    for k in range(_BB):
        _racnn_one(locs_ref, img_ref, out_ref, g * _BB + k, k)


def kernel(images, locs):
    imgs2 = images.reshape(_B, _C * _S, _S)
    return pl.pallas_call(
        _racnn_body,
        grid=(_B // _BB,),
        in_specs=[
            pl.BlockSpec(memory_space=pltpu.SMEM),
            pl.BlockSpec((_BB, _C * _S, _S), lambda b: (b, 0, 0)),
        ],
        out_specs=pl.BlockSpec((_BB, _C, _OUT, _OUT), lambda b: (b, 0, 0, 0)),
        out_shape=jax.ShapeDtypeStruct((_B, _C, _OUT, _OUT), jnp.float32),
        compiler_params=pltpu.CompilerParams(
            dimension_semantics=("arbitrary",),
        ),
    )(locs, imgs2)
